# baseline (device time: 387471 ns/iter reference)
import jax
import jax.numpy as jnp
from jax import lax
from jax.experimental import pallas as pl
from jax.experimental.pallas import tpu as pltpu

N_DEV = 16
M = 4096
N_OUT = 2048
CHUNK = M // N_DEV
N_SEG = 8
SEG = N_OUT // N_SEG
N_RS = N_DEV - 1
N_STEPS = 2 * (N_DEV - 1)
N_SLOTS = 8


def kernel(x, w_mat):
    def body(x_ref, w_ref, out_ref, *scratch):
        rbufs = scratch[0:N_SEG]
        ssems = scratch[N_SEG:2 * N_SEG]
        rsems = scratch[2 * N_SEG:3 * N_SEG]
        creds = scratch[3 * N_SEG:4 * N_SEG]

        my = lax.axis_index("i")
        left = (my - 1) % N_DEV
        right = (my + 1) % N_DEV

        half = N_SEG // 2
        rings = []
        for k in range(half):
            rings.append(
                (k * SEG, True, rbufs[k], ssems[k], rsems[k], creds[k])
            )
            rings.append(
                ((half + k) * SEG, False, rbufs[half + k],
                 ssems[half + k], rsems[half + k], creds[half + k])
            )

        barrier_sem = pltpu.get_barrier_semaphore()
        for nbr in (left, right):
            pl.semaphore_signal(
                barrier_sem, inc=1,
                device_id=(nbr,), device_id_type=pl.DeviceIdType.MESH,
            )
        pl.semaphore_wait(barrier_sem, 2)

        def rows(c):
            return pl.ds(c * CHUNK, CHUNK)

        def compute_chunk(c):
            out_ref[rows(c), :] = jnp.dot(
                x_ref[rows(c), :], w_ref[:, :],
                preferred_element_type=jnp.float32,
            )

        def chunk_ids(u, is_cw):
            if u < N_RS:
                if is_cw:
                    return (my - u) % N_DEV, (my - u - 1) % N_DEV
                return (my + u) % N_DEV, (my + u + 1) % N_DEV
            t = u - N_RS
            if is_cw:
                return (my + 1 - t) % N_DEV, (my - t) % N_DEV
            return (my - 1 + t) % N_DEV, (my + t) % N_DEV

        def make_desc(u, off, is_cw, rbuf, ssem, rsem):
            send_c, _recv_c = chunk_ids(u, is_cw)
            if u < N_RS:
                dst = rbuf.at[u % N_SLOTS]
            else:
                dst = out_ref.at[rows(send_c), pl.ds(off, SEG)]
            return pltpu.make_async_remote_copy(
                src_ref=out_ref.at[rows(send_c), pl.ds(off, SEG)],
                dst_ref=dst,
                send_sem=ssem.at[u],
                recv_sem=rsem.at[u],
                device_id=(right if is_cw else left,),
                device_id_type=pl.DeviceIdType.MESH,
            )

        compute_chunk(my)

        descs = {off: {} for off, *_ in rings}
        for off, is_cw, rbuf, ssem, rsem, _cred in rings:
            d = make_desc(0, off, is_cw, rbuf, ssem, rsem)
            descs[off][0] = d
            d.start()

        for s in range(N_STEPS):
            if s <= 6:
                compute_chunk((my - s - 1) % N_DEV)
                compute_chunk((my + s + 1) % N_DEV)
            elif s == 7:
                compute_chunk((my + 8) % N_DEV)

            for off, is_cw, rbuf, ssem, rsem, cred in rings:
                _send_c, recv_c = chunk_ids(s, is_cw)
                descs[off][s].wait_recv()
                if s < N_RS:
                    out_ref[rows(recv_c), pl.ds(off, SEG)] = (
                        out_ref[rows(recv_c), pl.ds(off, SEG)]
                        + rbuf[s % N_SLOTS]
                    )
                    if s < N_RS - N_SLOTS:
                        pl.semaphore_signal(
                            cred, inc=1,
                            device_id=(left if is_cw else right,),
                            device_id_type=pl.DeviceIdType.MESH,
                        )
                if s + 1 < N_STEPS:
                    u = s + 1
                    if N_SLOTS <= u < N_RS:
                        pl.semaphore_wait(cred, 1)
                    d = make_desc(u, off, is_cw, rbuf, ssem, rsem)
                    descs[off][u] = d
                    d.start()

        for off, *_ in rings:
            for u in range(N_STEPS):
                descs[off][u].wait_send()

    return pl.pallas_call(
        body,
        out_shape=jax.ShapeDtypeStruct((M, N_OUT), jnp.float32),
        in_specs=[
            pl.BlockSpec(memory_space=pltpu.VMEM),
            pl.BlockSpec(memory_space=pltpu.VMEM),
        ],
        out_specs=pl.BlockSpec(memory_space=pltpu.VMEM),
        scratch_shapes=(
            [pltpu.VMEM((N_SLOTS, CHUNK, SEG), jnp.float32)] * N_SEG
            + [pltpu.SemaphoreType.DMA((N_STEPS,))] * N_SEG
            + [pltpu.SemaphoreType.DMA((N_STEPS,))] * N_SEG
            + [pltpu.SemaphoreType.REGULAR] * N_SEG
        ),
        compiler_params=pltpu.CompilerParams(
            collective_id=0, vmem_limit_bytes=128 * 1024 * 1024
        ),
    )(x, w_mat)


# device time: 383064 ns/iter; 1.0115x vs baseline; 1.0115x over previous
import jax
import jax.numpy as jnp
from jax import lax
from jax.experimental import pallas as pl
from jax.experimental.pallas import tpu as pltpu

N_DEV = 16
M = 4096
N_OUT = 2048
CHUNK = M // N_DEV
SUB = CHUNK // 4
N_STEPS = 2 * (N_DEV - 1)


def kernel(x, w_mat):
    def body(
        x_ref, w_ref, out_ref,
        rbuf0, rbuf1, rbuf2, rbuf3,
        ssem0, ssem1, ssem2, ssem3,
        rsem0, rsem1, rsem2, rsem3,
        cred0, cred1, cred2, cred3,
    ):
        my = lax.axis_index("i")
        left = (my - 1) % N_DEV
        right = (my + 1) % N_DEV

        rings = [
            (0 * SUB, True, rbuf0, ssem0, rsem0, cred0),
            (2 * SUB, False, rbuf2, ssem2, rsem2, cred2),
            (1 * SUB, True, rbuf1, ssem1, rsem1, cred1),
            (3 * SUB, False, rbuf3, ssem3, rsem3, cred3),
        ]

        barrier_sem = pltpu.get_barrier_semaphore()
        for nbr in (left, right):
            pl.semaphore_signal(
                barrier_sem, inc=1,
                device_id=(nbr,), device_id_type=pl.DeviceIdType.MESH,
            )
        pl.semaphore_wait(barrier_sem, 2)

        def rows(c):
            return pl.ds(c * CHUNK, CHUNK)

        def compute_chunk(c):
            out_ref[rows(c), :] = jnp.dot(
                x_ref[rows(c), :], w_ref[:, :],
                preferred_element_type=jnp.float32,
            )

        def chunk_ids(u, is_cw):
            if u < N_DEV - 1:
                if is_cw:
                    return (my - u) % N_DEV, (my - u - 1) % N_DEV
                return (my + u) % N_DEV, (my + u + 1) % N_DEV
            t = u - (N_DEV - 1)
            if is_cw:
                return (my + 1 - t) % N_DEV, (my - t) % N_DEV
            return (my - 1 + t) % N_DEV, (my + t) % N_DEV

        def seg_rows(c, off):
            return pl.ds(c * CHUNK + off, SUB)

        def make_desc(u, off, is_cw, rbuf, ssem, rsem):
            slot = u % 2
            send_c, recv_c = chunk_ids(u, is_cw)
            if u < N_DEV - 1:
                dst = rbuf.at[slot]
            else:
                dst = out_ref.at[seg_rows(send_c, off), :]
            return pltpu.make_async_remote_copy(
                src_ref=out_ref.at[seg_rows(send_c, off), :],
                dst_ref=dst,
                send_sem=ssem.at[slot],
                recv_sem=rsem.at[slot],
                device_id=(right if is_cw else left,),
                device_id_type=pl.DeviceIdType.MESH,
            )

        compute_chunk(my)

        descs = {off: {} for off, *_ in rings}
        for off, is_cw, rbuf, ssem, rsem, _cred in rings:
            d = make_desc(0, off, is_cw, rbuf, ssem, rsem)
            descs[off][0] = d
            d.start()

        for s in range(N_STEPS):
            if s <= 6:
                compute_chunk((my - s - 1) % N_DEV)
                compute_chunk((my + s + 1) % N_DEV)
            elif s == 7:
                compute_chunk((my + 8) % N_DEV)

            for off, is_cw, rbuf, ssem, rsem, cred in rings:
                slot = s % 2
                _send_c, recv_c = chunk_ids(s, is_cw)
                descs[off][s].wait_recv()
                if s < N_DEV - 1:
                    out_ref[seg_rows(recv_c, off), :] = (
                        out_ref[seg_rows(recv_c, off), :] + rbuf[slot]
                    )
                if s < N_STEPS - 2:
                    pl.semaphore_signal(
                        cred, inc=1,
                        device_id=(left if is_cw else right,),
                        device_id_type=pl.DeviceIdType.MESH,
                    )
                if s + 1 < N_STEPS:
                    u = s + 1
                    if u >= 2:
                        pl.semaphore_wait(cred, 1)
                        descs[off][u - 2].wait_send()
                    d = make_desc(u, off, is_cw, rbuf, ssem, rsem)
                    descs[off][u] = d
                    d.start()

        for off, *_ in rings:
            descs[off][N_STEPS - 2].wait_send()
            descs[off][N_STEPS - 1].wait_send()

    return pl.pallas_call(
        body,
        out_shape=jax.ShapeDtypeStruct((M, N_OUT), jnp.float32),
        in_specs=[
            pl.BlockSpec(memory_space=pltpu.VMEM),
            pl.BlockSpec(memory_space=pltpu.VMEM),
        ],
        out_specs=pl.BlockSpec(memory_space=pltpu.VMEM),
        scratch_shapes=[
            pltpu.VMEM((2, SUB, N_OUT), jnp.float32),
            pltpu.VMEM((2, SUB, N_OUT), jnp.float32),
            pltpu.VMEM((2, SUB, N_OUT), jnp.float32),
            pltpu.VMEM((2, SUB, N_OUT), jnp.float32),
            pltpu.SemaphoreType.DMA((2,)),
            pltpu.SemaphoreType.DMA((2,)),
            pltpu.SemaphoreType.DMA((2,)),
            pltpu.SemaphoreType.DMA((2,)),
            pltpu.SemaphoreType.DMA((2,)),
            pltpu.SemaphoreType.DMA((2,)),
            pltpu.SemaphoreType.DMA((2,)),
            pltpu.SemaphoreType.DMA((2,)),
            pltpu.SemaphoreType.REGULAR,
            pltpu.SemaphoreType.REGULAR,
            pltpu.SemaphoreType.REGULAR,
            pltpu.SemaphoreType.REGULAR,
        ],
        compiler_params=pltpu.CompilerParams(
            collective_id=0, vmem_limit_bytes=100 * 1024 * 1024
        ),
    )(x, w_mat)
